# Initial kernel scaffold; baseline (speedup 1.0000x reference)
import functools

import jax
import jax.numpy as jnp
from jax import lax
from jax.experimental import pallas as pl
from jax.experimental.pallas import tpu as pltpu
from jax.experimental.pallas import tpu_sc as plsc

B = 16384
H = 50
E = 32
OUT = 128

NC = 2
NS = 16
NW = NC * NS
BAGS_PER_W = B // NW
CB = 16
NCHUNK = BAGS_PER_W // CB


def _pool_body(idx_hbm, table_hbm, out_hbm, idx_v, rows_v, acc_v, sem0, sem1):
    wid = lax.axis_index("s") * NC + lax.axis_index("c")
    base_bag = wid * BAGS_PER_W

    sems = (sem0, sem1)

    def start_gather(b, gg):
        pltpu.sync_copy(idx_hbm.at[pl.ds(base_bag + gg * CB, CB), :],
                        idx_v.at[b])
        pltpu.async_copy(table_hbm.at[idx_v.at[b]], rows_v.at[b], sems[b])

    def wait_gather(b):
        pltpu.make_async_copy(table_hbm.at[idx_v.at[b]], rows_v.at[b],
                              sems[b]).wait()

    def reduce_chunk(b, gg):
        for i in range(CB):
            a0 = rows_v[b, i, 0, 0:16]
            a1 = rows_v[b, i, 0, 16:32]
            for l in range(1, H):
                a0 = a0 + rows_v[b, i, l, 0:16]
                a1 = a1 + rows_v[b, i, l, 16:32]
            acc_v[i, 0:16] = a0
            acc_v[i, 16:32] = a1
        pltpu.sync_copy(acc_v,
                        out_hbm.at[pl.ds(base_bag + gg * CB, CB), :])

    start_gather(0, 0)

    def outer(g2, carry):
        for b in range(2):
            gg = g2 * 2 + b

            @pl.when(gg + 1 < NCHUNK)
            def _():
                start_gather(1 - b, gg + 1)

            wait_gather(b)
            reduce_chunk(b, gg)
        return carry

    lax.fori_loop(0, NCHUNK // 2, outer, 0)


_mesh = plsc.VectorSubcoreMesh(core_axis_name="c", subcore_axis_name="s")

_pool = functools.partial(
    pl.kernel,
    out_type=jax.ShapeDtypeStruct((B, E), jnp.float32),
    mesh=_mesh,
    scratch_types=[
        pltpu.VMEM((2, CB, H), jnp.int32),
        pltpu.VMEM((2, CB, H, E), jnp.float32),
        pltpu.VMEM((CB, E), jnp.float32),
        pltpu.SemaphoreType.DMA,
        pltpu.SemaphoreType.DMA,
    ],
)(_pool_body)


def _matmul_body(p_ref, w_ref, o_ref):
    o_ref[...] = lax.dot_general(
        p_ref[...], w_ref[...],
        (((1,), (1,)), ((), ())),
        preferred_element_type=jnp.float32,
    )


TB = 1024


def _project(pooled, W):
    return pl.pallas_call(
        _matmul_body,
        grid=(B // TB,),
        in_specs=[
            pl.BlockSpec((TB, E), lambda i: (i, 0)),
            pl.BlockSpec((OUT, E), lambda i: (0, 0)),
        ],
        out_specs=pl.BlockSpec((TB, OUT), lambda i: (i, 0)),
        out_shape=jax.ShapeDtypeStruct((B, OUT), jnp.float32),
    )(pooled, W)


@jax.jit
def kernel(input, table, W):
    pooled = _pool(input.astype(jnp.int32), table)
    return _project(pooled, W)


# speedup vs baseline: 2.3524x; 2.3524x over previous
import functools

import jax
import jax.numpy as jnp
from jax import lax
from jax.experimental import pallas as pl
from jax.experimental.pallas import tpu as pltpu
from jax.experimental.pallas import tpu_sc as plsc

B = 16384
H = 50
E = 32
OUT = 128

NC = 2
NS = 16
NW = NC * NS
BAGS_PER_W = B // NW
CB = 16
NCHUNK = BAGS_PER_W // CB
GL = 100
NG = CB * H // GL


def _pool_body(idx_hbm, table_hbm, out_hbm, idx_v, rows_v, acc_v, sem0, sem1):
    wid = lax.axis_index("s") * NC + lax.axis_index("c")
    base_bag = wid * BAGS_PER_W

    sems = (sem0, sem1)

    def start_gather(b, gg):
        pltpu.sync_copy(
            idx_hbm.at[pl.ds(pl.multiple_of((base_bag + gg * CB) * H // GL, 8),
                             NG), :],
            idx_v.at[b])
        for j in range(NG):
            pltpu.async_copy(table_hbm.at[idx_v.at[b, j]],
                             rows_v.at[b, j], sems[b])

    def wait_gather(b):
        for j in range(NG):
            pltpu.make_async_copy(table_hbm.at[idx_v.at[b, j]],
                                  rows_v.at[b, j], sems[b]).wait()

    def reduce_chunk(b, gg):
        for i in range(CB):
            j, r = (i * H) // GL, (i * H) % GL
            a0 = rows_v[b, j, r, 0:16]
            a1 = rows_v[b, j, r, 16:32]
            for l in range(1, H):
                j, r = (i * H + l) // GL, (i * H + l) % GL
                a0 = a0 + rows_v[b, j, r, 0:16]
                a1 = a1 + rows_v[b, j, r, 16:32]
            acc_v[i, 0:16] = a0
            acc_v[i, 16:32] = a1
        pltpu.sync_copy(acc_v,
                        out_hbm.at[pl.ds(base_bag + gg * CB, CB), :])

    start_gather(0, 0)

    def outer(g2, carry):
        for b in range(2):
            gg = g2 * 2 + b

            @pl.when(gg + 1 < NCHUNK)
            def _():
                start_gather(1 - b, gg + 1)

            wait_gather(b)
            reduce_chunk(b, gg)
        return carry

    lax.fori_loop(0, NCHUNK // 2, outer, 0)


_mesh = plsc.VectorSubcoreMesh(core_axis_name="c", subcore_axis_name="s")

_pool = functools.partial(
    pl.kernel,
    out_type=jax.ShapeDtypeStruct((B, E), jnp.float32),
    mesh=_mesh,
    compiler_params=pltpu.CompilerParams(use_tc_tiling_on_sc=False),
    scratch_types=[
        pltpu.VMEM((2, NG, GL), jnp.int32),
        pltpu.VMEM((2, NG, GL, E), jnp.float32),
        pltpu.VMEM((CB, E), jnp.float32),
        pltpu.SemaphoreType.DMA,
        pltpu.SemaphoreType.DMA,
    ],
)(_pool_body)


def _matmul_body(p_ref, w_ref, o_ref):
    o_ref[...] = lax.dot_general(
        p_ref[...], w_ref[...],
        (((1,), (1,)), ((), ())),
        preferred_element_type=jnp.float32,
    )


TB = 1024


def _project(pooled, W):
    return pl.pallas_call(
        _matmul_body,
        grid=(B // TB,),
        in_specs=[
            pl.BlockSpec((TB, E), lambda i: (i, 0)),
            pl.BlockSpec((OUT, E), lambda i: (0, 0)),
        ],
        out_specs=pl.BlockSpec((TB, OUT), lambda i: (i, 0)),
        out_shape=jax.ShapeDtypeStruct((B, OUT), jnp.float32),
    )(pooled, W)


@jax.jit
def kernel(input, table, W):
    pooled = _pool(input.astype(jnp.int32).reshape(B * H // GL, GL), table)
    return _project(pooled, W)
